# baseline (speedup 1.0000x reference)
import functools
import math

import jax
import jax.numpy as jnp
from jax import lax
from jax.experimental import pallas as pl
from jax.experimental.pallas import tpu as pltpu
from jax.experimental.pallas import tpu_sc as plsc

N = 4096
V = 32000
_S = 0.1 / (V - 1)
_CONF = 0.9
_C_ROW = 0.1 * math.log(_S) + _CONF * math.log(_CONF)

_RT = 3072
_L = 16

_BR = 1024
_BC = 6400
_NRB = _RT // _BR
_NCB = V // _BC


def _tc_body(x_ref, a_ref, b_ref, mall_ref, t_ref, out_ref, acc_ref):
    i = pl.program_id(0)
    j = pl.program_id(1)

    @pl.when((i == 0) & (j == 0))
    def _init():
        acc_ref[0] = 0.0
        acc_ref[1] = jnp.sum(mall_ref[...])

    col = lax.broadcasted_iota(jnp.int32, (_BR, _BC), 1) + j * _BC
    w = jnp.where(col == t_ref[...], b_ref[...], a_ref[...])
    acc_ref[0] += jnp.sum(x_ref[...] * w)

    @pl.when((i == _NRB - 1) & (j == _NCB - 1))
    def _final():
        row = lax.broadcasted_iota(jnp.int32, (8, 128), 0)
        out_ref[...] = jnp.where(row == 1, acc_ref[1], acc_ref[0])


def _tc_weighted_sum(x, a, b, maskf, tgt2d, interpret=False):
    rowspec = pl.BlockSpec((_BR, 1), lambda i, j: (i, 0))
    return pl.pallas_call(
        _tc_body,
        grid=(_NRB, _NCB),
        in_specs=[
            pl.BlockSpec((_BR, _BC), lambda i, j: (i, j)),
            rowspec, rowspec,
            pl.BlockSpec((N, 1), lambda i, j: (0, 0)),
            rowspec,
        ],
        out_specs=pl.BlockSpec((8, 128), lambda i, j: (0, 0)),
        out_shape=jax.ShapeDtypeStruct((8, 128), jnp.float32),
        scratch_shapes=[pltpu.SMEM((2,), jnp.float32)],
        interpret=interpret,
    )(x, a, b, maskf, tgt2d)


_NW = 32
_RPW = (N - _RT) // _NW
_CW = 1280
_NCH = (V // _CW) * (_RPW // 8)
_CPB = V // _CW


@functools.cache
def _sc_stream_fn():
    mesh = plsc.VectorSubcoreMesh(core_axis_name="c", subcore_axis_name="s")

    def _chunk_offsets(wid, k):
        band = k // _CPB
        r0 = _RT + wid * _RPW + band * 8
        c0 = (k % _CPB) * _CW
        return band, r0, c0

    @functools.partial(
        pl.kernel,
        mesh=mesh,
        compiler_params=pltpu.CompilerParams(needs_layout_passes=False),
        out_type=jax.ShapeDtypeStruct((_NW, _L), jnp.float32),
        scratch_types=[
            pltpu.VMEM((8, _CW), jnp.float32),
            pltpu.VMEM((8, _CW), jnp.float32),
            pltpu.VMEM((_RPW,), jnp.int32),
            pltpu.VMEM((_RPW,), jnp.float32),
            pltpu.VMEM((_RPW,), jnp.float32),
            pltpu.VMEM((_L,), jnp.float32),
            pltpu.SemaphoreType.DMA,
            pltpu.SemaphoreType.DMA,
        ],
    )
    def _sc_stream(x_hbm, tgt_hbm, arow_hbm, bma_hbm, out_hbm,
                   buf0, buf1, tgt_v, arow_v, bma_v, acc_v, sem0, sem1):
        wid = lax.axis_index("s") * 2 + lax.axis_index("c")
        rbase = _RT + wid * _RPW
        pltpu.sync_copy(tgt_hbm.at[pl.ds(rbase, _RPW)], tgt_v)
        pltpu.sync_copy(arow_hbm.at[pl.ds(rbase, _RPW)], arow_v)
        pltpu.sync_copy(bma_hbm.at[pl.ds(rbase, _RPW)], bma_v)

        iota = lax.iota(jnp.int32, _L)
        bufs = (buf0, buf1)
        sems = (sem0, sem1)

        def _start(wid_, k, buf, sem):
            band, r0, c0 = _chunk_offsets(wid_, k)
            pltpu.make_async_copy(
                x_hbm.at[pl.ds(r0, 8), pl.ds(c0, _CW)], buf, sem).start()

        def _wait(buf, sem):
            pltpu.make_async_copy(
                x_hbm.at[pl.ds(0, 8), pl.ds(0, _CW)], buf, sem).wait()

        def _compute(wid_, k, buf, acc):
            band, r0, c0 = _chunk_offsets(wid_, k)
            for r in range(8):
                ab = plsc.load_gather(arow_v, [jnp.full((_L,), band * 8 + r,
                                                        jnp.int32)])
                for v in range(_CW // _L):
                    acc = acc + buf[r, pl.ds(v * _L, _L)] * ab
            off = jnp.where(band == 3, 16, band * 8)
            lane_lo = band * 8 - off
            tl = plsc.load_gather(tgt_v, [off + iota])
            bl = plsc.load_gather(bma_v, [off + iota])
            valid = ((iota >= lane_lo) & (iota < lane_lo + 8)
                     & (tl >= c0) & (tl < c0 + _CW))
            rowi = jnp.where(valid, iota - lane_lo, 0)
            coli = jnp.where(valid, tl - c0, 0)
            xt = plsc.load_gather(buf, [rowi, coli], mask=valid)
            acc = acc + jnp.where(valid, xt * bl, 0.0)
            return acc

        _start(wid, 0, buf0, sem0)

        def _pair(p, acc):
            k0 = 2 * p
            _wait(buf0, sem0)
            _start(wid, k0 + 1, buf1, sem1)
            acc = _compute(wid, k0, buf0, acc)
            _wait(buf1, sem1)

            @pl.when(p < _NCH // 2 - 1)
            def _():
                _start(wid, k0 + 2, buf0, sem0)

            acc = _compute(wid, k0 + 1, buf1, acc)
            return acc

        acc = lax.fori_loop(0, _NCH // 2, _pair, jnp.zeros((_L,), jnp.float32))
        acc_v[...] = acc
        pltpu.sync_copy(acc_v, out_hbm.at[wid])

    return _sc_stream


@functools.cache
def _sc_finish_fn():
    mesh = plsc.VectorSubcoreMesh(core_axis_name="c", subcore_axis_name="s")

    @functools.partial(
        pl.kernel,
        mesh=mesh,
        compiler_params=pltpu.CompilerParams(needs_layout_passes=False),
        out_type=jax.ShapeDtypeStruct((_L,), jnp.float32),
        scratch_types=[
            pltpu.VMEM((8, 128), jnp.float32),
            pltpu.VMEM((_NW, _L), jnp.float32),
            pltpu.VMEM((_L,), jnp.float32),
            pltpu.VMEM((_L,), jnp.float32),
        ],
    )
    def _sc_finish(tcacc_hbm, scacc_hbm, out_hbm, tc_v, sc_v, rot_v, out_v):
        wid = lax.axis_index("s") * 2 + lax.axis_index("c")

        @pl.when(wid == 0)
        def _():
            pltpu.sync_copy(tcacc_hbm, tc_v)
            pltpu.sync_copy(scacc_hbm, sc_v)
            tot = jnp.zeros((_L,), jnp.float32)
            for w in range(_NW):
                tot = tot + sc_v[w, pl.ds(0, _L)]
            iota = lax.iota(jnp.int32, _L)
            for sh in (8, 4, 2, 1):
                rot_v[...] = tot
                tot = tot + plsc.load_gather(rot_v, [(iota + sh) & 15])
            s_tc = tc_v[0, pl.ds(0, _L)]
            m_cnt = tc_v[1, pl.ds(0, _L)]
            out_v[...] = m_cnt * _C_ROW - s_tc - tot
            pltpu.sync_copy(out_v, out_hbm)

    return _sc_finish


def kernel(x, target, target_mask):
    maskf = target_mask.astype(jnp.float32).reshape(N, 1)
    a = maskf * jnp.float32(_S)
    b = maskf * jnp.float32(_CONF)
    tgt2d = target.astype(jnp.int32).reshape(N, 1)
    arow = target_mask.astype(jnp.float32) * jnp.float32(_S)
    bma = target_mask.astype(jnp.float32) * jnp.float32(_CONF - _S)
    tgt1d = target.astype(jnp.int32)
    scacc = _sc_stream_fn()(x, tgt1d, arow, bma)
    tcacc = _tc_weighted_sum(x, a, b, maskf, tgt2d)
    out = _sc_finish_fn()(tcacc, scacc)
    return out[0]


# speedup vs baseline: 1.2118x; 1.2118x over previous
import functools
import math

import jax
import jax.numpy as jnp
from jax import lax
from jax.experimental import pallas as pl
from jax.experimental.pallas import tpu as pltpu
from jax.experimental.pallas import tpu_sc as plsc

N = 4096
V = 32000
_S = 0.1 / (V - 1)
_CONF = 0.9
_C_ROW = 0.1 * math.log(_S) + _CONF * math.log(_CONF)

_RT = 3072
_L = 16

_BR = 1024
_BC = 6400
_NRB = _RT // _BR
_NCB = V // _BC


def _tc_body(x_ref, a_ref, b_ref, mall_ref, t_ref, out_ref, acc_ref):
    i = pl.program_id(0)
    j = pl.program_id(1)

    @pl.when((i == 0) & (j == 0))
    def _init():
        acc_ref[0] = 0.0
        acc_ref[1] = jnp.sum(mall_ref[...])

    col = lax.broadcasted_iota(jnp.int32, (_BR, _BC), 1) + j * _BC
    w = jnp.where(col == t_ref[...], b_ref[...], a_ref[...])
    acc_ref[0] += jnp.sum(x_ref[...] * w)

    @pl.when((i == _NRB - 1) & (j == _NCB - 1))
    def _final():
        row = lax.broadcasted_iota(jnp.int32, (8, 128), 0)
        out_ref[...] = jnp.where(row == 1, acc_ref[1], acc_ref[0])


def _tc_weighted_sum(x, a, b, maskf, tgt2d, interpret=False):
    rowspec = pl.BlockSpec((_BR, 1), lambda i, j: (i, 0))
    return pl.pallas_call(
        _tc_body,
        grid=(_NRB, _NCB),
        in_specs=[
            pl.BlockSpec((_BR, _BC), lambda i, j: (i, j)),
            rowspec, rowspec,
            pl.BlockSpec((N, 1), lambda i, j: (0, 0)),
            rowspec,
        ],
        out_specs=pl.BlockSpec((8, 128), lambda i, j: (0, 0)),
        out_shape=jax.ShapeDtypeStruct((8, 128), jnp.float32),
        scratch_shapes=[pltpu.SMEM((2,), jnp.float32)],
        interpret=interpret,
    )(x, a, b, maskf, tgt2d)


_NW = 32
_RPW = (N - _RT) // _NW
_CW = 3200
_NCH = (V // _CW) * (_RPW // 8)
_CPB = V // _CW


@functools.cache
def _sc_stream_fn():
    mesh = plsc.VectorSubcoreMesh(core_axis_name="c", subcore_axis_name="s")

    def _chunk_offsets(wid, k):
        band = k // _CPB
        r0 = _RT + wid * _RPW + band * 8
        c0 = (k % _CPB) * _CW
        return band, r0, c0

    @functools.partial(
        pl.kernel,
        mesh=mesh,
        compiler_params=pltpu.CompilerParams(needs_layout_passes=False),
        out_type=jax.ShapeDtypeStruct((_NW, _L), jnp.float32),
        scratch_types=[
            pltpu.VMEM((8, _CW), jnp.float32),
            pltpu.VMEM((8, _CW), jnp.float32),
            pltpu.VMEM((_RPW,), jnp.int32),
            pltpu.VMEM((_RPW,), jnp.float32),
            pltpu.VMEM((_RPW,), jnp.float32),
            pltpu.VMEM((_L,), jnp.float32),
            pltpu.SemaphoreType.DMA,
            pltpu.SemaphoreType.DMA,
        ],
    )
    def _sc_stream(x_hbm, tgt_hbm, arow_hbm, bma_hbm, out_hbm,
                   buf0, buf1, tgt_v, arow_v, bma_v, acc_v, sem0, sem1):
        wid = lax.axis_index("s") * 2 + lax.axis_index("c")
        rbase = _RT + wid * _RPW
        pltpu.sync_copy(tgt_hbm.at[pl.ds(rbase, _RPW)], tgt_v)
        pltpu.sync_copy(arow_hbm.at[pl.ds(rbase, _RPW)], arow_v)
        pltpu.sync_copy(bma_hbm.at[pl.ds(rbase, _RPW)], bma_v)

        iota = lax.iota(jnp.int32, _L)
        bufs = (buf0, buf1)
        sems = (sem0, sem1)

        def _start(wid_, k, buf, sem):
            band, r0, c0 = _chunk_offsets(wid_, k)
            pltpu.make_async_copy(
                x_hbm.at[pl.ds(r0, 8), pl.ds(c0, _CW)], buf, sem).start()

        def _wait(buf, sem):
            pltpu.make_async_copy(
                x_hbm.at[pl.ds(0, 8), pl.ds(0, _CW)], buf, sem).wait()

        def _compute(wid_, k, buf, accs):
            acc0, acc1 = accs
            band, r0, c0 = _chunk_offsets(wid_, k)
            for r in range(8):
                ab = plsc.load_gather(arow_v, [jnp.full((_L,), band * 8 + r,
                                                        jnp.int32)])
                for v in range(_CW // _L):
                    if v % 2 == 0:
                        acc0 = acc0 + buf[r, pl.ds(v * _L, _L)] * ab
                    else:
                        acc1 = acc1 + buf[r, pl.ds(v * _L, _L)] * ab
            off = jnp.where(band == 3, 16, band * 8)
            lane_lo = band * 8 - off
            tl = plsc.load_gather(tgt_v, [off + iota])
            bl = plsc.load_gather(bma_v, [off + iota])
            valid = ((iota >= lane_lo) & (iota < lane_lo + 8)
                     & (tl >= c0) & (tl < c0 + _CW))
            rowi = jnp.where(valid, iota - lane_lo, 0)
            coli = jnp.where(valid, tl - c0, 0)
            xt = plsc.load_gather(buf, [rowi, coli], mask=valid)
            acc0 = acc0 + jnp.where(valid, xt * bl, 0.0)
            return acc0, acc1

        _start(wid, 0, buf0, sem0)

        def _pair(p, accs):
            k0 = 2 * p
            _wait(buf0, sem0)
            _start(wid, k0 + 1, buf1, sem1)
            accs = _compute(wid, k0, buf0, accs)
            _wait(buf1, sem1)

            @pl.when(p < _NCH // 2 - 1)
            def _():
                _start(wid, k0 + 2, buf0, sem0)

            accs = _compute(wid, k0 + 1, buf1, accs)
            return accs

        zero = jnp.zeros((_L,), jnp.float32)
        acc0, acc1 = lax.fori_loop(0, _NCH // 2, _pair, (zero, zero))
        acc_v[...] = acc0 + acc1
        pltpu.sync_copy(acc_v, out_hbm.at[wid])

    return _sc_stream


@functools.cache
def _sc_finish_fn():
    mesh = plsc.VectorSubcoreMesh(core_axis_name="c", subcore_axis_name="s")

    @functools.partial(
        pl.kernel,
        mesh=mesh,
        compiler_params=pltpu.CompilerParams(needs_layout_passes=False),
        out_type=jax.ShapeDtypeStruct((_L,), jnp.float32),
        scratch_types=[
            pltpu.VMEM((8, 128), jnp.float32),
            pltpu.VMEM((_NW, _L), jnp.float32),
            pltpu.VMEM((_L,), jnp.float32),
            pltpu.VMEM((_L,), jnp.float32),
        ],
    )
    def _sc_finish(tcacc_hbm, scacc_hbm, out_hbm, tc_v, sc_v, rot_v, out_v):
        wid = lax.axis_index("s") * 2 + lax.axis_index("c")

        @pl.when(wid == 0)
        def _():
            pltpu.sync_copy(tcacc_hbm, tc_v)
            pltpu.sync_copy(scacc_hbm, sc_v)
            tot = jnp.zeros((_L,), jnp.float32)
            for w in range(_NW):
                tot = tot + sc_v[w, pl.ds(0, _L)]
            iota = lax.iota(jnp.int32, _L)
            for sh in (8, 4, 2, 1):
                rot_v[...] = tot
                tot = tot + plsc.load_gather(rot_v, [(iota + sh) & 15])
            s_tc = tc_v[0, pl.ds(0, _L)]
            m_cnt = tc_v[1, pl.ds(0, _L)]
            out_v[...] = m_cnt * _C_ROW - s_tc - tot
            pltpu.sync_copy(out_v, out_hbm)

    return _sc_finish


def kernel(x, target, target_mask):
    maskf = target_mask.astype(jnp.float32).reshape(N, 1)
    a = maskf * jnp.float32(_S)
    b = maskf * jnp.float32(_CONF)
    tgt2d = target.astype(jnp.int32).reshape(N, 1)
    arow = target_mask.astype(jnp.float32) * jnp.float32(_S)
    bma = target_mask.astype(jnp.float32) * jnp.float32(_CONF - _S)
    tgt1d = target.astype(jnp.int32)
    scacc = _sc_stream_fn()(x, tgt1d, arow, bma)
    tcacc = _tc_weighted_sum(x, a, b, maskf, tgt2d)
    out = _sc_finish_fn()(tcacc, scacc)
    return out[0]
